# Initial kernel scaffold; baseline (speedup 1.0000x reference)
import jax
import jax.numpy as jnp
from jax.experimental import pallas as pl


def kernel(x, token, edge_attr, edge_index, edge_attr_T, W1, b1, W2, b2):
    raise NotImplementedError("write your pallas kernel here")


# speedup vs baseline: 4.1463x; 4.1463x over previous
import jax
import jax.numpy as jnp
from jax import lax
from jax.experimental import pallas as pl
from jax.experimental.pallas import tpu as pltpu
from jax.experimental.pallas import tpu_sc as plsc

N, E, D, DE, H = 10000, 320000, 128, 16, 10
NPAD = 10240
NBLK = 1024
EBLK = 2560
NC, NS = 2, 16
NW = NC * NS
EPW = E // NW
EB = 2000
GRPB = EB // 16


def _pack_rows(lo, hi):
    lo_u = lax.bitcast_convert_type(lo.astype(jnp.bfloat16), jnp.uint16)
    hi_u = lax.bitcast_convert_type(hi.astype(jnp.bfloat16), jnp.uint16)
    word = lo_u.astype(jnp.uint32) | (hi_u.astype(jnp.uint32) << 16)
    return lax.bitcast_convert_type(word, jnp.int32)


def _nodeproj_body(w_ref, x_ref, o_ref):
    m = lax.dot_general(
        w_ref[...], x_ref[...], (((0,), (1,)), ((), ())),
        preferred_element_type=jnp.float32)
    o_ref[...] = _pack_rows(m[0:16], m[16:32])


def _edgeproj_body(w_ref, b_ref, ea_ref, eat_ref, o_ref):
    w = w_ref[...]
    b = b_ref[:, 0:1]
    c1 = lax.dot_general(w, ea_ref[...], (((0,), (1,)), ((), ())),
                         preferred_element_type=jnp.float32) + b
    c2 = lax.dot_general(w, eat_ref[...], (((0,), (1,)), ((), ())),
                         preferred_element_type=jnp.float32) + b
    o_ref[...] = _pack_rows(c1, c2)


def _lo_f(w):
    return plsc.bitcast(lax.shift_left(w, 16), jnp.float32)


def _hi_f(w):
    return plsc.bitcast(w, jnp.float32)


def _sc_body(ntp_hbm, cpk_hbm, src_hbm, tgt_hbm, cst_hbm, out_hbm,
             ntp_v, src_v, tgt_v, cpk_v, out_v, cst_v):
    wid = lax.axis_index("s") * NC + lax.axis_index("c")
    base = wid * EPW
    pltpu.sync_copy(ntp_hbm, ntp_v)
    pltpu.sync_copy(cst_hbm, cst_v)

    for blk in range(EPW // EB):
        boff = base + blk * EB
        pltpu.sync_copy(src_hbm.at[pl.ds(boff, EB)], src_v)
        pltpu.sync_copy(tgt_hbm.at[pl.ds(boff, EB)], tgt_v)
        for j in range(H):
            pltpu.sync_copy(cpk_hbm.at[pl.ds(j * E + boff, EB)],
                            cpk_v.at[pl.ds(j * EB, EB)])

        def g_body(g, _):
            sl = pl.ds(g * 16, 16)
            si = src_v[sl]
            ti = tgt_v[sl]
            acc = cst_v[pl.ds(H * 16, 16)]
            for j in range(H):
                row = ntp_v.at[pl.ds(j * NPAD, NPAD)]
                ws = plsc.load_gather(row, [si])
                wt = plsc.load_gather(row, [ti])
                cw = cpk_v[pl.ds(g * 16 + j * EB, 16)]
                z1 = _lo_f(ws) + _hi_f(wt) + _lo_f(cw)
                z2 = _lo_f(wt) + _hi_f(ws) + _hi_f(cw)
                wj = cst_v[pl.ds(j * 16, 16)]
                acc = acc + (jnp.maximum(z1, 0.0)
                             + jnp.maximum(z2, 0.0)) * wj
            out_v[sl] = 1.0 / (1.0 + jnp.exp(-acc))
            return 0

        lax.fori_loop(0, GRPB, g_body, 0, unroll=2)
        pltpu.sync_copy(out_v, out_hbm.at[pl.ds(boff, EB)])


def kernel(x, token, edge_attr, edge_index, edge_attr_T, W1, b1, W2, b2):
    del token
    xp = jnp.pad(x, ((0, NPAD - N), (0, 0)))
    w1p = jnp.zeros((D, 32), jnp.float32)
    w1p = w1p.at[:, :H].set(W1[:D]).at[:, 16:16 + H].set(W1[D:2 * D])
    w1e = jnp.zeros((DE, 16), jnp.float32).at[:, :H].set(W1[2 * D:])
    b1p = jnp.broadcast_to(jnp.pad(b1, (0, 16 - H))[:, None], (16, 128))
    consts = (jnp.zeros((16,), jnp.float32)
              .at[:H].set(0.5 * (W2[:, 1] - W2[:, 0]))
              .at[H].set(b2[1] - b2[0]))
    ctile = jnp.broadcast_to(consts[:, None], (16, 16)).reshape(-1)

    ntp = pl.pallas_call(
        _nodeproj_body,
        grid=(NPAD // NBLK,),
        in_specs=[pl.BlockSpec((D, 32), lambda i: (0, 0)),
                  pl.BlockSpec((NBLK, D), lambda i: (i, 0))],
        out_specs=pl.BlockSpec((16, NBLK), lambda i: (0, i)),
        out_shape=jax.ShapeDtypeStruct((16, NPAD), jnp.int32),
    )(w1p, xp)

    cpk = pl.pallas_call(
        _edgeproj_body,
        grid=(E // EBLK,),
        in_specs=[pl.BlockSpec((DE, 16), lambda i: (0, 0)),
                  pl.BlockSpec((16, 128), lambda i: (0, 0)),
                  pl.BlockSpec((EBLK, DE), lambda i: (i, 0)),
                  pl.BlockSpec((EBLK, DE), lambda i: (i, 0))],
        out_specs=pl.BlockSpec((16, EBLK), lambda i: (0, i)),
        out_shape=jax.ShapeDtypeStruct((16, E), jnp.int32),
    )(w1e, b1p, edge_attr, edge_attr_T)

    mesh = plsc.VectorSubcoreMesh(core_axis_name="c", subcore_axis_name="s",
                                  num_cores=NC, num_subcores=NS)
    out = pl.kernel(
        _sc_body,
        out_type=jax.ShapeDtypeStruct((E,), jnp.float32),
        mesh=mesh,
        compiler_params=pltpu.CompilerParams(needs_layout_passes=False),
        scratch_types=[
            pltpu.VMEM((H * NPAD,), jnp.int32),
            pltpu.VMEM((EB,), jnp.int32),
            pltpu.VMEM((EB,), jnp.int32),
            pltpu.VMEM((H * EB,), jnp.int32),
            pltpu.VMEM((EB,), jnp.float32),
            pltpu.VMEM((256,), jnp.float32),
        ],
    )(ntp.reshape(-1)[:H * NPAD], cpk.reshape(-1), edge_index[0],
      edge_index[1], ctile)

    return out.reshape(E, 1)


# speedup vs baseline: 4.5072x; 1.0870x over previous
import jax
import jax.numpy as jnp
from jax import lax
from jax.experimental import pallas as pl
from jax.experimental.pallas import tpu as pltpu
from jax.experimental.pallas import tpu_sc as plsc

N, E, D, DE, H = 10000, 320000, 128, 16, 10
NPAD = 10240
NBLK = 1024
EBLK = 2560
NC, NS = 2, 16
NW = NC * NS
EPW = E // NW
EB = 400
NBLKS = EPW // EB
GRPB = EB // 16


def _pack_pair(lo, hi):
    lo_u = lax.bitcast_convert_type(lo.astype(jnp.bfloat16), jnp.uint16)
    hi_u = lax.bitcast_convert_type(hi.astype(jnp.bfloat16), jnp.uint16)
    word = lo_u.astype(jnp.uint32) | (hi_u.astype(jnp.uint32) << 16)
    return lax.bitcast_convert_type(word, jnp.int32)


def _nodeproj_body(w_ref, x_ref, o_ref):
    m = lax.dot_general(
        w_ref[...], x_ref[...], (((0,), (1,)), ((), ())),
        preferred_element_type=jnp.float32)
    o_ref[...] = _pack_pair(m[0:16], m[16:32])


def _edgeproj_body(w_ref, b_ref, ea_ref, eat_ref, o_ref):
    w = w_ref[...]
    b = b_ref[0:1, :]
    c1 = lax.dot_general(ea_ref[...], w, (((1,), (0,)), ((), ())),
                         preferred_element_type=jnp.float32) + b
    c2 = lax.dot_general(eat_ref[...], w, (((1,), (0,)), ((), ())),
                         preferred_element_type=jnp.float32) + b
    o_ref[...] = _pack_pair(c1, c2)


def _lo_f(w):
    return plsc.bitcast(lax.shift_left(w, 16), jnp.float32)


def _hi_f(w):
    return plsc.bitcast(w, jnp.float32)


def _sc_body(ntp_hbm, cpk_hbm, src_hbm, tgt_hbm, cst_hbm, out_hbm,
             ntp_v, src_v, tgt_v, cpk_v, out_v, cst_v):
    wid = lax.axis_index("s") * NC + lax.axis_index("c")
    base = wid * EPW
    pltpu.sync_copy(ntp_hbm, ntp_v)
    pltpu.sync_copy(cst_hbm, cst_v)
    pltpu.sync_copy(src_hbm.at[pl.ds(base, EPW)], src_v)
    pltpu.sync_copy(tgt_hbm.at[pl.ds(base, EPW)], tgt_v)

    def blk_body(blk, _):
        boff = base + blk * EB
        pltpu.sync_copy(cpk_hbm.at[pl.ds(boff * 16, EB * 16)], cpk_v)

        def g_body(g, _):
            eoff = blk * EB + g * 16
            si = src_v[pl.ds(eoff, 16)]
            ti = tgt_v[pl.ds(eoff, 16)]
            sv = lax.iota(jnp.int32, 16) * 16 + g * 256
            acc = cst_v[pl.ds(H * 16, 16)]
            for j in range(H):
                row = ntp_v.at[pl.ds(j * NPAD, NPAD)]
                ws = plsc.load_gather(row, [si])
                wt = plsc.load_gather(row, [ti])
                cw = plsc.load_gather(cpk_v, [sv + j])
                z1 = _lo_f(ws) + _hi_f(wt) + _lo_f(cw)
                z2 = _lo_f(wt) + _hi_f(ws) + _hi_f(cw)
                wj = cst_v[pl.ds(j * 16, 16)]
                acc = acc + (jnp.maximum(z1, 0.0)
                             + jnp.maximum(z2, 0.0)) * wj
            out_v[pl.ds(g * 16, 16)] = 1.0 / (1.0 + jnp.exp(-acc))
            return 0

        lax.fori_loop(0, GRPB, g_body, 0, unroll=2)
        pltpu.sync_copy(out_v, out_hbm.at[pl.ds(boff, EB)])
        return 0

    lax.fori_loop(0, NBLKS, blk_body, 0)


def kernel(x, token, edge_attr, edge_index, edge_attr_T, W1, b1, W2, b2):
    del token
    xp = jnp.pad(x, ((0, NPAD - N), (0, 0)))
    w1p = jnp.zeros((D, 32), jnp.float32)
    w1p = w1p.at[:, :H].set(W1[:D]).at[:, 16:16 + H].set(W1[D:2 * D])
    w1e = jnp.zeros((DE, 16), jnp.float32).at[:, :H].set(W1[2 * D:])
    wb = jnp.kron(jnp.eye(8, dtype=jnp.float32), w1e)
    b1bc = jnp.broadcast_to(
        jnp.tile(jnp.pad(b1, (0, 16 - H)), 8)[None, :], (8, 128))
    consts = (jnp.zeros((16,), jnp.float32)
              .at[:H].set(0.5 * (W2[:, 1] - W2[:, 0]))
              .at[H].set(b2[1] - b2[0]))
    ctile = jnp.broadcast_to(consts[:, None], (16, 16)).reshape(-1)
    ea8 = edge_attr.reshape(E // 8, 8 * DE)
    eat8 = edge_attr_T.reshape(E // 8, 8 * DE)

    ntp = pl.pallas_call(
        _nodeproj_body,
        grid=(NPAD // NBLK,),
        in_specs=[pl.BlockSpec((D, 32), lambda i: (0, 0)),
                  pl.BlockSpec((NBLK, D), lambda i: (i, 0))],
        out_specs=pl.BlockSpec((16, NBLK), lambda i: (0, i)),
        out_shape=jax.ShapeDtypeStruct((16, NPAD), jnp.int32),
    )(w1p, xp)

    cpk = pl.pallas_call(
        _edgeproj_body,
        grid=(E // EBLK,),
        in_specs=[pl.BlockSpec((8 * DE, 8 * DE), lambda i: (0, 0)),
                  pl.BlockSpec((8, 8 * DE), lambda i: (0, 0)),
                  pl.BlockSpec((EBLK // 8, 8 * DE), lambda i: (i, 0)),
                  pl.BlockSpec((EBLK // 8, 8 * DE), lambda i: (i, 0))],
        out_specs=pl.BlockSpec((EBLK // 8, 8 * DE), lambda i: (i, 0)),
        out_shape=jax.ShapeDtypeStruct((E // 8, 8 * DE), jnp.int32),
    )(wb, b1bc, ea8, eat8)

    mesh = plsc.VectorSubcoreMesh(core_axis_name="c", subcore_axis_name="s",
                                  num_cores=NC, num_subcores=NS)
    out = pl.kernel(
        _sc_body,
        out_type=jax.ShapeDtypeStruct((E,), jnp.float32),
        mesh=mesh,
        compiler_params=pltpu.CompilerParams(needs_layout_passes=False),
        scratch_types=[
            pltpu.VMEM((H * NPAD,), jnp.int32),
            pltpu.VMEM((EPW,), jnp.int32),
            pltpu.VMEM((EPW,), jnp.int32),
            pltpu.VMEM((EB * 16,), jnp.int32),
            pltpu.VMEM((EB,), jnp.float32),
            pltpu.VMEM((256,), jnp.float32),
        ],
    )(ntp.reshape(-1)[:H * NPAD], cpk.reshape(-1), edge_index[0],
      edge_index[1], ctile)

    return out.reshape(E, 1)
